# baseline (speedup 1.0000x reference)
import functools

import jax
import jax.numpy as jnp
from jax import lax
from jax.experimental import pallas as pl
from jax.experimental.pallas import tpu as pltpu
from jax.experimental.pallas import tpu_sc as plsc

_LANES = 16
_FILL_ROWS = 64
_COPY_ROWS = 16


def _build_sc_kernel(L, rows_per_layer, D, num_workers, nc):
    rows_per_worker = rows_per_layer // num_workers
    n_fill = rows_per_worker // _FILL_ROWS
    n_copy = rows_per_worker // _COPY_ROWS
    mesh = plsc.VectorSubcoreMesh(core_axis_name="c", subcore_axis_name="s")

    @functools.partial(
        pl.kernel,
        mesh=mesh,
        out_type=jax.ShapeDtypeStruct((L, rows_per_layer, D), jnp.float32),
        scratch_types=[
            pltpu.VMEM((_LANES,), jnp.int32),
            pltpu.VMEM((_FILL_ROWS, D), jnp.float32),
            pltpu.VMEM((_COPY_ROWS, D), jnp.float32),
            pltpu.VMEM((_COPY_ROWS, D), jnp.float32),
            pltpu.SemaphoreType.DMA,
            pltpu.SemaphoreType.DMA,
            pltpu.SemaphoreType.DMA,
        ],
    )
    def body(cond_hbm, mask_hbm, null_hbm, out_hbm,
             mask_v, null_buf, stage_a, stage_b, sem_w, sem_a, sem_b):
        wid = lax.axis_index("s") * nc + lax.axis_index("c")
        start = wid * rows_per_worker

        cp_m = pltpu.async_copy(mask_hbm, mask_v, sem_a)
        cp_n = pltpu.async_copy(null_hbm, null_buf.at[0], sem_b)
        cp_m.wait()
        cp_n.wait()

        @pl.loop(1, _FILL_ROWS)
        def _replicate(r):
            for j in range(D // _LANES):
                v = null_buf[0, pl.ds(j * _LANES, _LANES)]
                null_buf[r, pl.ds(j * _LANES, _LANES)] = v

        mvec = mask_v[...]
        m_all = mvec[0]
        for l in range(1, L):
            m_all = m_all & mvec[l]

        def _layer(l):
            m_l = mvec[l]

            @pl.when(m_l != 0)
            def _fill(l=l):
                fills = []
                for c in range(n_fill):
                    dst = out_hbm.at[l, pl.ds(start + c * _FILL_ROWS,
                                              _FILL_ROWS)]
                    fills.append(pltpu.async_copy(null_buf, dst, sem_w))
                for cp in fills:
                    cp.wait()

            @pl.when(m_l == 0)
            def _copy(l=l):
                def src(c):
                    return cond_hbm.at[l, pl.ds(start + c * _COPY_ROWS,
                                                _COPY_ROWS)]

                def dst(c):
                    return out_hbm.at[l, pl.ds(start + c * _COPY_ROWS,
                                               _COPY_ROWS)]

                bufs = (stage_a, stage_b)
                sems = (sem_a, sem_b)
                loads = [None, None]
                stores = [None, None]
                loads[0] = pltpu.async_copy(src(0), bufs[0], sems[0])
                for c in range(n_copy):
                    p = c % 2
                    q = (c + 1) % 2
                    if c + 1 < n_copy:
                        if stores[q] is not None:
                            stores[q].wait()
                        loads[q] = pltpu.async_copy(src(c + 1), bufs[q],
                                                    sems[q])
                    loads[p].wait()
                    stores[p] = pltpu.async_copy(bufs[p], dst(c), sems[p])
                for st in stores:
                    if st is not None:
                        st.wait()

        @pl.when(m_all != 0)
        def _fill_all():
            fills = []
            for l in range(L):
                for c in range(n_fill):
                    dst = out_hbm.at[l, pl.ds(start + c * _FILL_ROWS,
                                              _FILL_ROWS)]
                    fills.append(pltpu.async_copy(null_buf, dst, sem_w))
            for cp in fills:
                cp.wait()

        @pl.when(m_all == 0)
        def _general():
            for l in range(L):
                _layer(l)

    return body


def kernel(cond, eval_dropout_mask, nullcond):
    L, B, N, D = cond.shape
    rows_per_layer = B * N
    info = plsc.get_sparse_core_info()
    nc, ns = info.num_cores, info.num_subcores

    cond2 = cond.reshape(L, rows_per_layer, D)
    mask_i32 = jnp.pad(eval_dropout_mask.astype(jnp.int32), (0, _LANES - L))

    sc = _build_sc_kernel(L, rows_per_layer, D, nc * ns, nc)
    out = sc(cond2, mask_i32, nullcond)
    return out.reshape(L, B, N, D)


# speedup vs baseline: 1.0574x; 1.0574x over previous
import functools

import jax
import jax.numpy as jnp
from jax import lax
from jax.experimental import pallas as pl
from jax.experimental.pallas import tpu as pltpu
from jax.experimental.pallas import tpu_sc as plsc

_LANES = 16
_FILL_ROWS = 32
_COPY_ROWS = 16


def _build_sc_kernel(L, rows_per_layer, D, num_workers, nc):
    rows_per_worker = rows_per_layer // num_workers
    n_fill = rows_per_worker // _FILL_ROWS
    n_copy = rows_per_worker // _COPY_ROWS
    mesh = plsc.VectorSubcoreMesh(core_axis_name="c", subcore_axis_name="s")

    @functools.partial(
        pl.kernel,
        mesh=mesh,
        out_type=jax.ShapeDtypeStruct((L, rows_per_layer, D), jnp.float32),
        scratch_types=[
            pltpu.VMEM((_LANES,), jnp.int32),
            pltpu.VMEM((_FILL_ROWS, D), jnp.float32),
            pltpu.VMEM((_COPY_ROWS, D), jnp.float32),
            pltpu.VMEM((_COPY_ROWS, D), jnp.float32),
            pltpu.SemaphoreType.DMA,
            pltpu.SemaphoreType.DMA,
            pltpu.SemaphoreType.DMA,
        ],
    )
    def body(cond_hbm, mask_hbm, null_hbm, out_hbm,
             mask_v, null_buf, stage_a, stage_b, sem_w, sem_a, sem_b):
        wid = lax.axis_index("s") * nc + lax.axis_index("c")
        start = wid * rows_per_worker

        cp_m = pltpu.async_copy(mask_hbm, mask_v, sem_a)
        cp_n = pltpu.async_copy(null_hbm, null_buf.at[0], sem_b)
        cp_m.wait()
        cp_n.wait()

        @pl.loop(1, _FILL_ROWS)
        def _replicate(r):
            for j in range(D // _LANES):
                v = null_buf[0, pl.ds(j * _LANES, _LANES)]
                null_buf[r, pl.ds(j * _LANES, _LANES)] = v

        mvec = mask_v[...]
        m_all = mvec[0]
        for l in range(1, L):
            m_all = m_all & mvec[l]

        def _layer(l):
            m_l = mvec[l]

            @pl.when(m_l != 0)
            def _fill(l=l):
                fills = []
                for c in range(n_fill):
                    dst = out_hbm.at[l, pl.ds(start + c * _FILL_ROWS,
                                              _FILL_ROWS)]
                    fills.append(pltpu.async_copy(null_buf, dst, sem_w))
                for cp in fills:
                    cp.wait()

            @pl.when(m_l == 0)
            def _copy(l=l):
                def src(c):
                    return cond_hbm.at[l, pl.ds(start + c * _COPY_ROWS,
                                                _COPY_ROWS)]

                def dst(c):
                    return out_hbm.at[l, pl.ds(start + c * _COPY_ROWS,
                                               _COPY_ROWS)]

                bufs = (stage_a, stage_b)
                sems = (sem_a, sem_b)
                loads = [None, None]
                stores = [None, None]
                loads[0] = pltpu.async_copy(src(0), bufs[0], sems[0])
                for c in range(n_copy):
                    p = c % 2
                    q = (c + 1) % 2
                    if c + 1 < n_copy:
                        if stores[q] is not None:
                            stores[q].wait()
                        loads[q] = pltpu.async_copy(src(c + 1), bufs[q],
                                                    sems[q])
                    loads[p].wait()
                    stores[p] = pltpu.async_copy(bufs[p], dst(c), sems[p])
                for st in stores:
                    if st is not None:
                        st.wait()

        @pl.when(m_all != 0)
        def _fill_all():
            fills = []
            for l in range(L):
                for c in range(n_fill):
                    dst = out_hbm.at[l, pl.ds(start + c * _FILL_ROWS,
                                              _FILL_ROWS)]
                    fills.append(pltpu.async_copy(null_buf, dst, sem_w))
            for cp in fills:
                cp.wait()

        @pl.when(m_all == 0)
        def _general():
            for l in range(L):
                _layer(l)

    return body


def kernel(cond, eval_dropout_mask, nullcond):
    L, B, N, D = cond.shape
    rows_per_layer = B * N
    info = plsc.get_sparse_core_info()
    nc, ns = info.num_cores, info.num_subcores

    cond2 = cond.reshape(L, rows_per_layer, D)
    mask_i32 = jnp.pad(eval_dropout_mask.astype(jnp.int32), (0, _LANES - L))

    sc = _build_sc_kernel(L, rows_per_layer, D, nc * ns, nc)
    out = sc(cond2, mask_i32, nullcond)
    return out.reshape(L, B, N, D)


# speedup vs baseline: 1.0783x; 1.0198x over previous
import functools

import jax
import jax.numpy as jnp
from jax import lax
from jax.experimental import pallas as pl
from jax.experimental.pallas import tpu as pltpu
from jax.experimental.pallas import tpu_sc as plsc

_LANES = 16
_FILL_ROWS = 16
_COPY_ROWS = 16


def _build_sc_kernel(L, rows_per_layer, D, num_workers, nc):
    rows_per_worker = rows_per_layer // num_workers
    n_fill = rows_per_worker // _FILL_ROWS
    n_copy = rows_per_worker // _COPY_ROWS
    mesh = plsc.VectorSubcoreMesh(core_axis_name="c", subcore_axis_name="s")

    @functools.partial(
        pl.kernel,
        mesh=mesh,
        out_type=jax.ShapeDtypeStruct((L, rows_per_layer, D), jnp.float32),
        scratch_types=[
            pltpu.VMEM((_LANES,), jnp.int32),
            pltpu.VMEM((_FILL_ROWS, D), jnp.float32),
            pltpu.VMEM((_COPY_ROWS, D), jnp.float32),
            pltpu.VMEM((_COPY_ROWS, D), jnp.float32),
            pltpu.SemaphoreType.DMA,
            pltpu.SemaphoreType.DMA,
            pltpu.SemaphoreType.DMA,
        ],
    )
    def body(cond_hbm, mask_hbm, null_hbm, out_hbm,
             mask_v, null_buf, stage_a, stage_b, sem_w, sem_a, sem_b):
        wid = lax.axis_index("s") * nc + lax.axis_index("c")
        start = wid * rows_per_worker

        cp_m = pltpu.async_copy(mask_hbm, mask_v, sem_a)
        cp_n = pltpu.async_copy(null_hbm, null_buf.at[0], sem_b)
        cp_m.wait()
        cp_n.wait()

        @pl.loop(1, _FILL_ROWS)
        def _replicate(r):
            for j in range(D // _LANES):
                v = null_buf[0, pl.ds(j * _LANES, _LANES)]
                null_buf[r, pl.ds(j * _LANES, _LANES)] = v

        mvec = mask_v[...]
        m_all = mvec[0]
        for l in range(1, L):
            m_all = m_all & mvec[l]

        def _layer(l):
            m_l = mvec[l]

            @pl.when(m_l != 0)
            def _fill(l=l):
                fills = []
                for c in range(n_fill):
                    dst = out_hbm.at[l, pl.ds(start + c * _FILL_ROWS,
                                              _FILL_ROWS)]
                    fills.append(pltpu.async_copy(null_buf, dst, sem_w))
                for cp in fills:
                    cp.wait()

            @pl.when(m_l == 0)
            def _copy(l=l):
                def src(c):
                    return cond_hbm.at[l, pl.ds(start + c * _COPY_ROWS,
                                                _COPY_ROWS)]

                def dst(c):
                    return out_hbm.at[l, pl.ds(start + c * _COPY_ROWS,
                                               _COPY_ROWS)]

                bufs = (stage_a, stage_b)
                sems = (sem_a, sem_b)
                loads = [None, None]
                stores = [None, None]
                loads[0] = pltpu.async_copy(src(0), bufs[0], sems[0])
                for c in range(n_copy):
                    p = c % 2
                    q = (c + 1) % 2
                    if c + 1 < n_copy:
                        if stores[q] is not None:
                            stores[q].wait()
                        loads[q] = pltpu.async_copy(src(c + 1), bufs[q],
                                                    sems[q])
                    loads[p].wait()
                    stores[p] = pltpu.async_copy(bufs[p], dst(c), sems[p])
                for st in stores:
                    if st is not None:
                        st.wait()

        @pl.when(m_all != 0)
        def _fill_all():
            fills = []
            for l in range(L):
                for c in range(n_fill):
                    dst = out_hbm.at[l, pl.ds(start + c * _FILL_ROWS,
                                              _FILL_ROWS)]
                    fills.append(pltpu.async_copy(null_buf, dst, sem_w))
            for cp in fills:
                cp.wait()

        @pl.when(m_all == 0)
        def _general():
            for l in range(L):
                _layer(l)

    return body


def kernel(cond, eval_dropout_mask, nullcond):
    L, B, N, D = cond.shape
    rows_per_layer = B * N
    info = plsc.get_sparse_core_info()
    nc, ns = info.num_cores, info.num_subcores

    cond2 = cond.reshape(L, rows_per_layer, D)
    mask_i32 = jnp.pad(eval_dropout_mask.astype(jnp.int32), (0, _LANES - L))

    sc = _build_sc_kernel(L, rows_per_layer, D, nc * ns, nc)
    out = sc(cond2, mask_i32, nullcond)
    return out.reshape(L, B, N, D)
